# baseline (speedup 1.0000x reference)
import functools

import jax
import jax.numpy as jnp
from jax import lax
from jax.experimental import pallas as pl
from jax.experimental.pallas import tpu as pltpu
from jax.experimental.pallas import tpu_sc as plsc

B, L = 1024, 200
V, P, T, H = 1000000, 512, 2, 128
N = B * L
NC, NS, LANES = 2, 16, 16
NW = NC * NS
ROWS_PER_W = N // NW
C = 128
CHUNKS = ROWS_PER_W // C
EPS = 1e-12


def _rsqrt(x):
    i = plsc.bitcast(x, jnp.int32)
    i = jnp.int32(0x5F3759DF) - lax.shift_right_arithmetic(i, 1)
    y = plsc.bitcast(i, jnp.float32)
    for _ in range(3):
        y = y * (1.5 - 0.5 * x * y * y)
    return y


def _sc_body(pid_hbm, eid_hbm, ent_hbm, pos_hbm, typ_hbm, gam_hbm, bet_hbm,
             out_hbm, eid_v, pid_v, ent_v, pos_v, out_v, gam_v, bet_v, typ_v,
             sem_e, sem_p):
    wid = lax.axis_index("s") * NC + lax.axis_index("c")
    iota = lax.iota(jnp.int32, LANES)

    pltpu.sync_copy(gam_hbm, gam_v)
    pltpu.sync_copy(bet_hbm, bet_v)
    pltpu.sync_copy(typ_hbm.at[0], typ_v)

    @pl.loop(0, CHUNKS)
    def _chunk(ch):
        base = wid * ROWS_PER_W + ch * C

        pltpu.sync_copy(eid_hbm.at[pl.ds(base, C)], eid_v)
        pltpu.sync_copy(pid_hbm.at[pl.ds(base, C)], pid_v)
        ge = pltpu.async_copy(ent_hbm.at[eid_v], ent_v, sem_e)
        gp = pltpu.async_copy(pos_hbm.at[pid_v], pos_v, sem_p)
        ge.wait()
        gp.wait()

        @pl.loop(0, C // LANES)
        def _block(b):
            rows = b * LANES + iota

            def pass1(j, carry):
                acc_s, acc_q = carry
                jj = jnp.full((LANES,), 0, jnp.int32) + j
                e = plsc.load_gather(ent_v, [rows, jj])
                p = plsc.load_gather(pos_v, [rows, jj])
                t = plsc.load_gather(typ_v, [jj])
                v = e + p + t
                plsc.store_scatter(out_v, [rows, jj], v)
                return acc_s + v, acc_q + v * v

            zero = jnp.zeros((LANES,), jnp.float32)
            acc_s, acc_q = pl.loop(0, H, init_carry=(zero, zero), unroll=4)(pass1)

            mu = acc_s * (1.0 / H)
            var = acc_q * (1.0 / H) - mu * mu
            rs = _rsqrt(var + EPS)

            @pl.loop(0, H, unroll=4)
            def pass2(j):
                jj = jnp.full((LANES,), 0, jnp.int32) + j
                v = plsc.load_gather(out_v, [rows, jj])
                g = plsc.load_gather(gam_v, [jj])
                bt = plsc.load_gather(bet_v, [jj])
                plsc.store_scatter(out_v, [rows, jj], (v - mu) * rs * g + bt)

        pltpu.sync_copy(out_v, out_hbm.at[pl.ds(base, C)])


@functools.partial(jax.jit, donate_argnums=())
def kernel(position_ids, entity_ids, entity_table, pos_table, type_table,
           ln_gamma, ln_beta):
    pid_flat = position_ids.reshape(N)
    eid_flat = entity_ids.reshape(N)

    mesh = plsc.VectorSubcoreMesh(core_axis_name="c", subcore_axis_name="s")
    f = pl.kernel(
        _sc_body,
        out_type=jax.ShapeDtypeStruct((N, H), jnp.float32),
        mesh=mesh,
        scratch_types=[
            pltpu.VMEM((C,), jnp.int32),
            pltpu.VMEM((C,), jnp.int32),
            pltpu.VMEM((C, H), jnp.float32),
            pltpu.VMEM((C, H), jnp.float32),
            pltpu.VMEM((C, H), jnp.float32),
            pltpu.VMEM((H,), jnp.float32),
            pltpu.VMEM((H,), jnp.float32),
            pltpu.VMEM((H,), jnp.float32),
            pltpu.SemaphoreType.DMA,
            pltpu.SemaphoreType.DMA,
        ],
        compiler_params=pltpu.CompilerParams(needs_layout_passes=False),
    )
    out = f(pid_flat, eid_flat, entity_table, pos_table, type_table,
            ln_gamma, ln_beta)
    return out.reshape(B, L, H)


# speedup vs baseline: 6.2967x; 6.2967x over previous
import functools

import jax
import jax.numpy as jnp
from jax import lax
from jax.experimental import pallas as pl
from jax.experimental.pallas import tpu as pltpu
from jax.experimental.pallas import tpu_sc as plsc

B, L = 1024, 200
V, P, T, H = 1000000, 512, 2, 128
N = B * L
NC, NS, LANES = 2, 16, 16
NW = NC * NS
ROWS_PER_W = N // NW
C = 128
CHUNKS = ROWS_PER_W // C
HV = H // LANES
EPS = 1e-12


def _bc(s):
    return lax.broadcast_in_dim(s, (LANES,), ())


def _rsqrt_scalar(x):
    i = lax.bitcast_convert_type(x, jnp.int32)
    i = jnp.int32(0x5F3759DF) - lax.shift_right_arithmetic(i, 1)
    y = lax.bitcast_convert_type(i, jnp.float32)
    for _ in range(3):
        y = y * (1.5 - 0.5 * x * y * y)
    return y


def _sc_body(pid_hbm, eid_hbm, ent_hbm, pos_hbm, typ_hbm, gam_hbm, bet_hbm,
             out_hbm, eid_v, pid_v, ent_a, pos_a, ent_b, pos_b, gam_v, bet_v,
             typ_v, sem_ea, sem_pa, sem_eb, sem_pb):
    wid = lax.axis_index("s") * NC + lax.axis_index("c")
    wbase = wid * ROWS_PER_W
    iota = lax.iota(jnp.int32, LANES)

    pltpu.sync_copy(eid_hbm.at[pl.ds(wbase, ROWS_PER_W)], eid_v)
    pltpu.sync_copy(pid_hbm.at[pl.ds(wbase, ROWS_PER_W)], pid_v)
    pltpu.sync_copy(gam_hbm, gam_v)
    pltpu.sync_copy(bet_hbm, bet_v)
    pltpu.sync_copy(typ_hbm.at[0], typ_v)

    tv = [typ_v[pl.ds(LANES * j, LANES)] for j in range(HV)]
    gv = [gam_v[pl.ds(LANES * j, LANES)] for j in range(HV)]
    bv = [bet_v[pl.ds(LANES * j, LANES)] for j in range(HV)]

    def start_gathers(ch, ent_buf, pos_buf, sem_e, sem_p):
        ge = pltpu.async_copy(ent_hbm.at[eid_v.at[pl.ds(ch * C, C)]],
                              ent_buf, sem_e)
        gp = pltpu.async_copy(pos_hbm.at[pid_v.at[pl.ds(ch * C, C)]],
                              pos_buf, sem_p)
        return ge, gp

    def wait_gathers(ent_buf, pos_buf, sem_e, sem_p):
        pltpu.make_async_copy(ent_hbm.at[eid_v.at[pl.ds(0, C)]],
                              ent_buf, sem_e).wait()
        pltpu.make_async_copy(pos_hbm.at[pid_v.at[pl.ds(0, C)]],
                              pos_buf, sem_p).wait()

    def compute_chunk(ch, ent_buf, pos_buf):
        cols = [iota + (LANES * j) for j in range(HV)]

        @pl.loop(0, C, unroll=2)
        def _row(r):
            rr = _bc(r)
            v = [plsc.load_gather(ent_buf, [rr, cols[j]])
                 + plsc.load_gather(pos_buf, [rr, cols[j]]) + tv[j]
                 for j in range(HV)]
            s = v[0] + v[1]
            s2 = v[2] + v[3]
            s3 = v[4] + v[5]
            s4 = v[6] + v[7]
            s = (s + s2) + (s3 + s4)
            q = v[0] * v[0] + v[1] * v[1]
            q2 = v[2] * v[2] + v[3] * v[3]
            q3 = v[4] * v[4] + v[5] * v[5]
            q4 = v[6] * v[6] + v[7] * v[7]
            q = (q + q2) + (q3 + q4)
            tot = jnp.sum(s)
            sq = jnp.sum(q)
            mu = tot * (1.0 / H)
            var = sq * (1.0 / H) - mu * mu
            rs = _rsqrt_scalar(var + EPS)
            mu_v = _bc(mu)
            rs_v = _bc(rs)
            for j in range(HV):
                o = (v[j] - mu_v) * rs_v * gv[j] + bv[j]
                plsc.store_scatter(ent_buf, [rr, cols[j]], o)

        pltpu.sync_copy(ent_buf, out_hbm.at[pl.ds(wbase + ch * C, C)])

    start_gathers(0, ent_a, pos_a, sem_ea, sem_pa)

    @pl.loop(0, CHUNKS, step=2)
    def _pipe(g):
        start_gathers(g + 1, ent_b, pos_b, sem_eb, sem_pb)
        wait_gathers(ent_a, pos_a, sem_ea, sem_pa)
        compute_chunk(g, ent_a, pos_a)

        @pl.when(g + 2 < CHUNKS)
        def _():
            start_gathers(g + 2, ent_a, pos_a, sem_ea, sem_pa)

        wait_gathers(ent_b, pos_b, sem_eb, sem_pb)
        compute_chunk(g + 1, ent_b, pos_b)


@functools.partial(jax.jit, donate_argnums=())
def kernel(position_ids, entity_ids, entity_table, pos_table, type_table,
           ln_gamma, ln_beta):
    pid_flat = position_ids.reshape(N)
    eid_flat = entity_ids.reshape(N)

    mesh = plsc.VectorSubcoreMesh(core_axis_name="c", subcore_axis_name="s")
    f = pl.kernel(
        _sc_body,
        out_type=jax.ShapeDtypeStruct((N, H), jnp.float32),
        mesh=mesh,
        scratch_types=[
            pltpu.VMEM((ROWS_PER_W,), jnp.int32),
            pltpu.VMEM((ROWS_PER_W,), jnp.int32),
            pltpu.VMEM((C, H), jnp.float32),
            pltpu.VMEM((C, H), jnp.float32),
            pltpu.VMEM((C, H), jnp.float32),
            pltpu.VMEM((C, H), jnp.float32),
            pltpu.VMEM((H,), jnp.float32),
            pltpu.VMEM((H,), jnp.float32),
            pltpu.VMEM((H,), jnp.float32),
            pltpu.SemaphoreType.DMA,
            pltpu.SemaphoreType.DMA,
            pltpu.SemaphoreType.DMA,
            pltpu.SemaphoreType.DMA,
        ],
        compiler_params=pltpu.CompilerParams(needs_layout_passes=False),
    )
    out = f(pid_flat, eid_flat, entity_table, pos_table, type_table,
            ln_gamma, ln_beta)
    return out.reshape(B, L, H)
